# baseline (speedup 1.0000x reference)
import functools

import jax
import jax.numpy as jnp
from jax import lax
from jax.experimental import pallas as pl
from jax.experimental.pallas import tpu as pltpu
from jax.experimental.pallas import tpu_sc as plsc

_BATCH = 16384
_EMB = 128
_NC = 2
_NS = 16
_NW = _NC * _NS
_BPW = _BATCH // _NW
_K = 64
_REPS = _BPW // _K
_LANES = _EMB // 16


def _sc_body(table_hbm, idx_hbm, out_hbm, idx_v, row1_v, rows_v, gsem, wsem):
    cid = lax.axis_index("c")
    sid = lax.axis_index("s")
    wid = sid * _NC + cid

    pltpu.sync_copy(idx_hbm, idx_v)
    pltpu.async_copy(table_hbm.at[idx_v], row1_v, gsem).wait()

    regs = [row1_v[0, pl.ds(j * 16, 16)] for j in range(_LANES)]
    for r in range(_K):
        for j in range(_LANES):
            rows_v[r, pl.ds(j * 16, 16)] = regs[j]

    base = wid * _BPW
    copies = [
        pltpu.async_copy(rows_v, out_hbm.at[pl.ds(base + j * _K, _K)], wsem)
        for j in range(_REPS)
    ]
    for c in copies:
        c.wait()


def kernel(pro, buf_grad, i):
    del pro
    idx = jnp.full((1,), i, dtype=jnp.int32)
    mesh = plsc.VectorSubcoreMesh(core_axis_name="c", subcore_axis_name="s")
    run = functools.partial(
        pl.kernel,
        out_type=jax.ShapeDtypeStruct((_BATCH, _EMB), jnp.float32),
        mesh=mesh,
        scratch_types=[
            pltpu.VMEM((1,), jnp.int32),
            pltpu.VMEM((1, _EMB), jnp.float32),
            pltpu.VMEM((_K, _EMB), jnp.float32),
            pltpu.SemaphoreType.DMA,
            pltpu.SemaphoreType.DMA,
        ],
    )(_sc_body)
    return run(buf_grad, idx)


# speedup vs baseline: 1.0813x; 1.0813x over previous
import functools

import jax
import jax.numpy as jnp
from jax import lax
from jax.experimental import pallas as pl
from jax.experimental.pallas import tpu as pltpu
from jax.experimental.pallas import tpu_sc as plsc

_BATCH = 16384
_EMB = 128
_G = 8
_BLK = 8192


def _sc_gather_body(table_hbm, idx_hbm, out_hbm, idx_v, row_v, sem):
    @pl.when((lax.axis_index("c") == 0) & (lax.axis_index("s") == 0))
    def _():
        pltpu.sync_copy(idx_hbm, idx_v)
        pltpu.async_copy(table_hbm.at[idx_v], row_v, sem).wait()
        pltpu.sync_copy(row_v, out_hbm)


def _tc_bcast_body(vec_ref, out_ref):
    out_ref[...] = jnp.broadcast_to(vec_ref[0:1], out_ref.shape)


def kernel(pro, buf_grad, i):
    del pro
    idx = jnp.full((_G,), i, dtype=jnp.int32)
    mesh = plsc.VectorSubcoreMesh(core_axis_name="c", subcore_axis_name="s")
    gather = functools.partial(
        pl.kernel,
        out_type=jax.ShapeDtypeStruct((_G, _EMB), jnp.float32),
        mesh=mesh,
        scratch_types=[
            pltpu.VMEM((_G,), jnp.int32),
            pltpu.VMEM((_G, _EMB), jnp.float32),
            pltpu.SemaphoreType.DMA,
        ],
    )(_sc_gather_body)
    staged = gather(buf_grad, idx)
    return pl.pallas_call(
        _tc_bcast_body,
        grid=(_BATCH // _BLK,),
        in_specs=[pl.BlockSpec((_G, _EMB), lambda g: (0, 0))],
        out_specs=pl.BlockSpec((_BLK, _EMB), lambda g: (g, 0)),
        out_shape=jax.ShapeDtypeStruct((_BATCH, _EMB), jnp.float32),
    )(staged)


# speedup vs baseline: 1.1017x; 1.0188x over previous
import functools

import jax
import jax.numpy as jnp
from jax import lax
from jax.experimental import pallas as pl
from jax.experimental.pallas import tpu as pltpu
from jax.experimental.pallas import tpu_sc as plsc

_BATCH = 16384
_EMB = 128
_NC = 2
_NS = 16
_NW = _NC * _NS
_BPW = _BATCH // _NW
_G = 8
_K = 32
_REPS = _BPW // _K
_LANES = _EMB // 16


def _sc_body(table_hbm, idx_hbm, out_hbm, idx_v, rows_v, row_spmem, gsem, wsem):
    cid = lax.axis_index("c")
    sid = lax.axis_index("s")
    wid = sid * _NC + cid

    @pl.when(sid == 0)
    def _gather_row():
        pltpu.sync_copy(idx_hbm, idx_v)
        pltpu.async_copy(table_hbm.at[idx_v], rows_v.at[pl.ds(0, _G)], gsem).wait()
        pltpu.sync_copy(rows_v.at[pl.ds(0, _G)], row_spmem)

    plsc.subcore_barrier()
    pltpu.sync_copy(row_spmem, rows_v.at[pl.ds(0, _G)])

    regs = [rows_v[0, pl.ds(j * 16, 16)] for j in range(_LANES)]
    for r in range(1, _K):
        for j in range(_LANES):
            rows_v[r, pl.ds(j * 16, 16)] = regs[j]

    base = wid * _BPW
    copies = [
        pltpu.async_copy(rows_v, out_hbm.at[pl.ds(base + j * _K, _K)], wsem)
        for j in range(_REPS)
    ]
    for c in copies:
        c.wait()


def kernel(pro, buf_grad, i):
    del pro
    idx = jnp.full((_G,), i, dtype=jnp.int32)
    mesh = plsc.VectorSubcoreMesh(core_axis_name="c", subcore_axis_name="s")
    run = functools.partial(
        pl.kernel,
        out_type=jax.ShapeDtypeStruct((_BATCH, _EMB), jnp.float32),
        mesh=mesh,
        scratch_types=[
            pltpu.VMEM((_G,), jnp.int32),
            pltpu.VMEM((_K, _EMB), jnp.float32),
            pltpu.VMEM_SHARED((_G, _EMB), jnp.float32),
            pltpu.SemaphoreType.DMA,
            pltpu.SemaphoreType.DMA,
        ],
    )(_sc_body)
    return run(buf_grad, idx)


# speedup vs baseline: 1.1122x; 1.0095x over previous
import functools

import jax
import jax.numpy as jnp
from jax import lax
from jax.experimental import pallas as pl
from jax.experimental.pallas import tpu as pltpu
from jax.experimental.pallas import tpu_sc as plsc

_BATCH = 16384
_EMB = 128
_NC = 2
_NS = 16
_NW = _NC * _NS
_BPW = _BATCH // _NW
_G = 8
_K = 16
_REPS = _BPW // _K
_LANES = _EMB // 16


def _sc_body(table_hbm, idx_hbm, out_hbm, idx_v, rows_v, row_spmem, gsem, wsem):
    cid = lax.axis_index("c")
    sid = lax.axis_index("s")
    wid = sid * _NC + cid

    @pl.when(sid == 0)
    def _gather_row():
        pltpu.sync_copy(idx_hbm, idx_v)
        pltpu.async_copy(table_hbm.at[idx_v], rows_v.at[pl.ds(0, _G)], gsem).wait()
        pltpu.sync_copy(rows_v.at[pl.ds(0, _G)], row_spmem)

    plsc.subcore_barrier()
    pltpu.sync_copy(row_spmem, rows_v.at[pl.ds(0, _G)])

    regs = [rows_v[0, pl.ds(j * 16, 16)] for j in range(_LANES)]
    for r in range(1, _K):
        for j in range(_LANES):
            rows_v[r, pl.ds(j * 16, 16)] = regs[j]

    base = wid * _BPW
    copies = [
        pltpu.async_copy(rows_v, out_hbm.at[pl.ds(base + j * _K, _K)], wsem)
        for j in range(_REPS)
    ]
    for c in copies:
        c.wait()


def kernel(pro, buf_grad, i):
    del pro
    idx = jnp.full((_G,), i, dtype=jnp.int32)
    mesh = plsc.VectorSubcoreMesh(core_axis_name="c", subcore_axis_name="s")
    run = functools.partial(
        pl.kernel,
        out_type=jax.ShapeDtypeStruct((_BATCH, _EMB), jnp.float32),
        mesh=mesh,
        scratch_types=[
            pltpu.VMEM((_G,), jnp.int32),
            pltpu.VMEM((_K, _EMB), jnp.float32),
            pltpu.VMEM_SHARED((_G, _EMB), jnp.float32),
            pltpu.SemaphoreType.DMA,
            pltpu.SemaphoreType.DMA,
        ],
    )(_sc_body)
    return run(buf_grad, idx)
